# baseline (speedup 1.0000x reference)
import functools

import jax
import jax.numpy as jnp
from jax import lax
from jax.experimental import pallas as pl
from jax.experimental.pallas import tpu as pltpu
from jax.experimental.pallas import tpu_sc as plsc

VOCAB = 100000
DMODEL = 1024
CTX = 2048
B = 4
SEQ = 2048

N = B * SEQ
NC = 2
NS = 16
NW = NC * NS
POS_PER_W = SEQ // NW
PC = 16
NPG = POS_PER_W // PC
LANES = 16
GROUPS = DMODEL // LANES

_TASKS = [(ph, b) for ph in range(NPG) for b in range(B)]

_mesh = plsc.VectorSubcoreMesh(
    core_axis_name="c", subcore_axis_name="s", num_cores=NC, num_subcores=NS
)


@functools.partial(
    pl.kernel,
    out_type=jax.ShapeDtypeStruct((N, DMODEL), jnp.float32),
    mesh=_mesh,
    scratch_types=[
        pltpu.VMEM((B * POS_PER_W,), jnp.int32),
        pltpu.VMEM((PC, DMODEL), jnp.float32),
        pltpu.VMEM((PC, DMODEL), jnp.float32),
        pltpu.VMEM((PC, DMODEL), jnp.float32),
        pltpu.VMEM((PC, DMODEL), jnp.float32),
        pltpu.VMEM((PC, DMODEL), jnp.float32),
        pltpu.VMEM((PC, DMODEL), jnp.float32),
        pltpu.VMEM((PC, DMODEL), jnp.float32),
        pltpu.SemaphoreType.DMA,
        pltpu.SemaphoreType.DMA,
        pltpu.SemaphoreType.DMA,
        pltpu.SemaphoreType.DMA,
        pltpu.SemaphoreType.DMA,
        pltpu.SemaphoreType.DMA,
        pltpu.SemaphoreType.DMA,
        pltpu.SemaphoreType.DMA,
        pltpu.SemaphoreType.DMA,
        pltpu.SemaphoreType.DMA,
        pltpu.SemaphoreType.DMA,
        pltpu.SemaphoreType.DMA,
    ],
)
def _embed(x_hbm, wt_hbm, pt_hbm, out_hbm,
           idx_v, w0, w1, w2, w3, w4, p0, p1,
           gs0, gs1, gs2, gs3, gs4, os0, os1, os2, os3, os4, ps0, ps1):
    wid = lax.axis_index("s") * NC + lax.axis_index("c")
    wpos = wid * POS_PER_W

    word = [w0, w1, w2, w3, w4]
    pos = [p0, p1]
    gsem = [gs0, gs1, gs2, gs3, gs4]
    osem = [os0, os1, os2, os3, os4]
    psem = [ps0, ps1]

    for b in range(B):
        pltpu.sync_copy(
            x_hbm.at[pl.ds(b * SEQ + wpos, POS_PER_W)],
            idx_v.at[pl.ds(b * POS_PER_W, POS_PER_W)],
        )

    NBUF = 5
    DEPTH = 3

    def start_gather(t):
        ph, b = _TASKS[t]
        buf = t % NBUF
        return pltpu.async_copy(
            wt_hbm.at[idx_v.at[pl.ds(b * POS_PER_W + ph * PC, PC)]],
            word[buf], gsem[buf],
        )

    def start_pos(ph):
        return pltpu.async_copy(
            pt_hbm.at[pl.ds(wpos + ph * PC, PC)], pos[ph % 2], psem[ph % 2]
        )

    pos_h = [None, None]
    out_h = [None] * NBUF
    gather_h = [None] * NBUF
    pos_h[0] = start_pos(0)
    for t in range(DEPTH):
        gather_h[t % NBUF] = start_gather(t)

    for t, (ph, b) in enumerate(_TASKS):
        cur = t % NBUF
        if b == 0 and ph + 1 < NPG:
            pos_h[(ph + 1) % 2] = start_pos(ph + 1)
        if t + DEPTH < len(_TASKS):
            nb = (t + DEPTH) % NBUF
            if out_h[nb] is not None:
                out_h[nb].wait()
                out_h[nb] = None
            gather_h[nb] = start_gather(t + DEPTH)
        gather_h[cur].wait()
        if b == 0:
            pos_h[ph % 2].wait()

        wbuf, pbuf = word[cur], pos[ph % 2]

        def add_body(k, carry):
            r = k // GROUPS
            g = k % GROUPS
            sl = pl.ds(g * LANES, LANES)
            plsc.addupdate(wbuf.at[r, sl], pbuf[r, sl])
            return carry

        lax.fori_loop(0, PC * GROUPS, add_body, 0, unroll=2)
        out_h[cur] = pltpu.async_copy(
            word[cur], out_hbm.at[pl.ds(b * SEQ + wpos + ph * PC, PC)],
            osem[cur],
        )

    for h in out_h:
        if h is not None:
            h.wait()


def kernel(x, word_table, pos_table):
    x_flat = x.reshape(N).astype(jnp.int32)
    out = _embed(x_flat, word_table, pos_table)
    return out.reshape(1, B, SEQ, DMODEL)


# speedup vs baseline: 1.1864x; 1.1864x over previous
import functools

import jax
import jax.numpy as jnp
from jax import lax
from jax.experimental import pallas as pl
from jax.experimental.pallas import tpu as pltpu
from jax.experimental.pallas import tpu_sc as plsc

VOCAB = 100000
DMODEL = 1024
CTX = 2048
B = 4
SEQ = 2048

N = B * SEQ
NC = 2
NS = 16
NW = NC * NS
POS_PER_W = SEQ // NW
PC = 16
NPG = POS_PER_W // PC
LANES = 16
GROUPS = DMODEL // LANES

_TASKS = [(ph, b) for ph in range(NPG) for b in range(B)]

_mesh = plsc.VectorSubcoreMesh(
    core_axis_name="c", subcore_axis_name="s", num_cores=NC, num_subcores=NS
)


@functools.partial(
    pl.kernel,
    out_type=jax.ShapeDtypeStruct((N, DMODEL), jnp.float32),
    mesh=_mesh,
    scratch_types=[
        pltpu.VMEM((B * POS_PER_W,), jnp.int32),
        pltpu.VMEM((PC, DMODEL), jnp.float32),
        pltpu.VMEM((PC, DMODEL), jnp.float32),
        pltpu.VMEM((PC, DMODEL), jnp.float32),
        pltpu.VMEM((PC, DMODEL), jnp.float32),
        pltpu.VMEM((PC, DMODEL), jnp.float32),
        pltpu.VMEM((PC, DMODEL), jnp.float32),
        pltpu.VMEM((PC, DMODEL), jnp.float32),
        pltpu.SemaphoreType.DMA,
        pltpu.SemaphoreType.DMA,
        pltpu.SemaphoreType.DMA,
        pltpu.SemaphoreType.DMA,
        pltpu.SemaphoreType.DMA,
        pltpu.SemaphoreType.DMA,
        pltpu.SemaphoreType.DMA,
        pltpu.SemaphoreType.DMA,
        pltpu.SemaphoreType.DMA,
        pltpu.SemaphoreType.DMA,
        pltpu.SemaphoreType.DMA,
        pltpu.SemaphoreType.DMA,
    ],
)
def _embed(x_hbm, wt_hbm, pt_hbm, out_hbm,
           idx_v, w0, w1, w2, w3, w4, p0, p1,
           gs0, gs1, gs2, gs3, gs4, os0, os1, os2, os3, os4, ps0, ps1):
    wid = lax.axis_index("s") * NC + lax.axis_index("c")
    wpos = wid * POS_PER_W

    word = [w0, w1, w2, w3, w4]
    pos = [p0, p1]
    gsem = [gs0, gs1, gs2, gs3, gs4]
    osem = [os0, os1, os2, os3, os4]
    psem = [ps0, ps1]

    for b in range(B):
        pltpu.sync_copy(
            x_hbm.at[pl.ds(b * SEQ + wpos, POS_PER_W)],
            idx_v.at[pl.ds(b * POS_PER_W, POS_PER_W)],
        )

    NBUF = 5
    DEPTH = 3

    def start_gather(t):
        ph, b = _TASKS[t]
        buf = t % NBUF
        return pltpu.async_copy(
            wt_hbm.at[idx_v.at[pl.ds(b * POS_PER_W + ph * PC, PC)]],
            word[buf], gsem[buf],
        )

    def start_pos(ph):
        return pltpu.async_copy(
            pt_hbm.at[pl.ds(wpos + ph * PC, PC)], pos[ph % 2], psem[ph % 2]
        )

    pos_h = [None, None]
    out_h = [None] * NBUF
    gather_h = [None] * NBUF
    pos_h[0] = start_pos(0)
    for t in range(DEPTH):
        gather_h[t % NBUF] = start_gather(t)

    for t, (ph, b) in enumerate(_TASKS):
        cur = t % NBUF
        if b == 0 and ph + 1 < NPG:
            pos_h[(ph + 1) % 2] = start_pos(ph + 1)
        if t + DEPTH < len(_TASKS):
            nb = (t + DEPTH) % NBUF
            if out_h[nb] is not None:
                out_h[nb].wait()
                out_h[nb] = None
            gather_h[nb] = start_gather(t + DEPTH)
        gather_h[cur].wait()
        if b == 0:
            pos_h[ph % 2].wait()

        wbuf, pbuf = word[cur], pos[ph % 2]

        def add_body(k, carry):
            r = k // GROUPS
            g = k % GROUPS
            sl = pl.ds(g * LANES, LANES)
            plsc.addupdate(wbuf.at[r, sl], pbuf[r, sl])
            return carry

        lax.fori_loop(0, PC * GROUPS, add_body, 0, unroll=4)
        out_h[cur] = pltpu.async_copy(
            word[cur], out_hbm.at[pl.ds(b * SEQ + wpos + ph * PC, PC)],
            osem[cur],
        )

    for h in out_h:
        if h is not None:
            h.wait()


def kernel(x, word_table, pos_table):
    x_flat = x.reshape(N).astype(jnp.int32)
    out = _embed(x_flat, word_table, pos_table)
    return out.reshape(1, B, SEQ, DMODEL)


# speedup vs baseline: 1.2142x; 1.0234x over previous
import functools

import jax
import jax.numpy as jnp
from jax import lax
from jax.experimental import pallas as pl
from jax.experimental.pallas import tpu as pltpu
from jax.experimental.pallas import tpu_sc as plsc

VOCAB = 100000
DMODEL = 1024
CTX = 2048
B = 4
SEQ = 2048

N = B * SEQ
NC = 2
NS = 16
NW = NC * NS
POS_PER_W = SEQ // NW
PC = 16
NPG = POS_PER_W // PC
LANES = 16
GROUPS = DMODEL // LANES

_TASKS = [(ph, b) for ph in range(NPG) for b in range(B)]

_mesh = plsc.VectorSubcoreMesh(
    core_axis_name="c", subcore_axis_name="s", num_cores=NC, num_subcores=NS
)


@functools.partial(
    pl.kernel,
    out_type=jax.ShapeDtypeStruct((N, DMODEL), jnp.float32),
    mesh=_mesh,
    scratch_types=[
        pltpu.VMEM((B * POS_PER_W,), jnp.int32),
        pltpu.VMEM((PC, DMODEL), jnp.float32),
        pltpu.VMEM((PC, DMODEL), jnp.float32),
        pltpu.VMEM((PC, DMODEL), jnp.float32),
        pltpu.VMEM((PC, DMODEL), jnp.float32),
        pltpu.VMEM((PC, DMODEL), jnp.float32),
        pltpu.VMEM((PC, DMODEL), jnp.float32),
        pltpu.VMEM((PC, DMODEL), jnp.float32),
        pltpu.SemaphoreType.DMA,
        pltpu.SemaphoreType.DMA,
        pltpu.SemaphoreType.DMA,
        pltpu.SemaphoreType.DMA,
        pltpu.SemaphoreType.DMA,
        pltpu.SemaphoreType.DMA,
        pltpu.SemaphoreType.DMA,
        pltpu.SemaphoreType.DMA,
        pltpu.SemaphoreType.DMA,
        pltpu.SemaphoreType.DMA,
        pltpu.SemaphoreType.DMA,
        pltpu.SemaphoreType.DMA,
        pltpu.SemaphoreType.DMA,
    ],
)
def _embed(x_hbm, wt_hbm, pt_hbm, out_hbm,
           idx_v, w0, w1, w2, w3, w4, p0, p1,
           gs0, gs1, gs2, gs3, gs4, os0, os1, os2, os3, os4, ps0, ps1, isem):
    wid = lax.axis_index("s") * NC + lax.axis_index("c")
    wpos = wid * POS_PER_W

    word = [w0, w1, w2, w3, w4]
    pos = [p0, p1]
    gsem = [gs0, gs1, gs2, gs3, gs4]
    osem = [os0, os1, os2, os3, os4]
    psem = [ps0, ps1]


    NBUF = 5
    DEPTH = 3

    def start_gather(t):
        ph, b = _TASKS[t]
        buf = t % NBUF
        return pltpu.async_copy(
            wt_hbm.at[idx_v.at[pl.ds(b * POS_PER_W + ph * PC, PC)]],
            word[buf], gsem[buf],
        )

    def start_pos(ph):
        return pltpu.async_copy(
            pt_hbm.at[pl.ds(wpos + ph * PC, PC)], pos[ph % 2], psem[ph % 2]
        )

    pos_h = [None, None]
    out_h = [None] * NBUF
    gather_h = [None] * NBUF
    pos_h[0] = start_pos(0)
    idx_h = []
    for b in range(B):
        idx_h.append(pltpu.async_copy(
            x_hbm.at[pl.ds(b * SEQ + wpos, POS_PER_W)],
            idx_v.at[pl.ds(b * POS_PER_W, POS_PER_W)],
            isem,
        ))
    for h in idx_h:
        h.wait()
    for t in range(DEPTH):
        gather_h[t % NBUF] = start_gather(t)

    for t, (ph, b) in enumerate(_TASKS):
        cur = t % NBUF
        if b == 0 and ph + 1 < NPG:
            pos_h[(ph + 1) % 2] = start_pos(ph + 1)
        if t + DEPTH < len(_TASKS):
            nb = (t + DEPTH) % NBUF
            if out_h[nb] is not None:
                out_h[nb].wait()
                out_h[nb] = None
            gather_h[nb] = start_gather(t + DEPTH)
        gather_h[cur].wait()
        if b == 0:
            pos_h[ph % 2].wait()

        wbuf, pbuf = word[cur], pos[ph % 2]

        def add_body(k, carry):
            r = k // GROUPS
            g = k % GROUPS
            sl = pl.ds(g * LANES, LANES)
            plsc.addupdate(wbuf.at[r, sl], pbuf[r, sl])
            return carry

        lax.fori_loop(0, PC * GROUPS, add_body, 0, unroll=4)
        out_h[cur] = pltpu.async_copy(
            word[cur], out_hbm.at[pl.ds(b * SEQ + wpos + ph * PC, PC)],
            osem[cur],
        )

    for h in out_h:
        if h is not None:
            h.wait()


def kernel(x, word_table, pos_table):
    x_flat = x.reshape(N).astype(jnp.int32)
    out = _embed(x_flat, word_table, pos_table)
    return out.reshape(1, B, SEQ, DMODEL)


# speedup vs baseline: 1.9635x; 1.6171x over previous
import functools

import jax
import jax.numpy as jnp
from jax import lax
from jax.experimental import pallas as pl
from jax.experimental.pallas import tpu as pltpu
from jax.experimental.pallas import tpu_sc as plsc

VOCAB = 100000
DMODEL = 1024
CTX = 2048
B = 4
SEQ = 2048

N = B * SEQ
NC = 2
NS = 16
NW = NC * NS
POS_PER_W = SEQ // NW
PC = 16
NPG = POS_PER_W // PC
LANES = 16
GROUPS = DMODEL // LANES

_TASKS = [(ph, b) for ph in range(NPG) for b in range(B)]

_mesh = plsc.VectorSubcoreMesh(
    core_axis_name="c", subcore_axis_name="s", num_cores=NC, num_subcores=NS
)


@functools.partial(
    pl.kernel,
    out_type=jax.ShapeDtypeStruct((N, DMODEL), jnp.float32),
    mesh=_mesh,
    scratch_types=[
        pltpu.VMEM((B * POS_PER_W,), jnp.int32),
        pltpu.VMEM((PC, DMODEL), jnp.float32),
        pltpu.VMEM((PC, DMODEL), jnp.float32),
        pltpu.VMEM((PC, DMODEL), jnp.float32),
        pltpu.VMEM((PC, DMODEL), jnp.float32),
        pltpu.VMEM((PC, DMODEL), jnp.float32),
        pltpu.VMEM((PC, DMODEL), jnp.float32),
        pltpu.VMEM((PC, DMODEL), jnp.float32),
        pltpu.SemaphoreType.DMA,
        pltpu.SemaphoreType.DMA,
        pltpu.SemaphoreType.DMA,
        pltpu.SemaphoreType.DMA,
        pltpu.SemaphoreType.DMA,
        pltpu.SemaphoreType.DMA,
        pltpu.SemaphoreType.DMA,
        pltpu.SemaphoreType.DMA,
        pltpu.SemaphoreType.DMA,
        pltpu.SemaphoreType.DMA,
        pltpu.SemaphoreType.DMA,
        pltpu.SemaphoreType.DMA,
        pltpu.SemaphoreType.DMA,
    ],
)
def _embed(x_hbm, wt_hbm, pt_hbm, out_hbm,
           idx_v, w0, w1, w2, w3, w4, p0, p1,
           gs0, gs1, gs2, gs3, gs4, os0, os1, os2, os3, os4, ps0, ps1, isem):
    wid = lax.axis_index("s") * NC + lax.axis_index("c")
    wpos = wid * POS_PER_W

    word = [w0, w1, w2, w3, w4]
    pos = [p0, p1]
    gsem = [gs0, gs1, gs2, gs3, gs4]
    osem = [os0, os1, os2, os3, os4]
    psem = [ps0, ps1]


    NBUF = 4
    DEPTH = 2

    def start_gather(t):
        ph, b = _TASKS[t]
        buf = t % NBUF
        return pltpu.async_copy(
            wt_hbm.at[idx_v.at[pl.ds(b * POS_PER_W + ph * PC, PC)]],
            word[buf], gsem[buf],
        )

    def start_pos(ph):
        return pltpu.async_copy(
            pt_hbm.at[pl.ds(wpos + ph * PC, PC)], pos[ph % 2], psem[ph % 2]
        )

    pos_h = [None, None]
    out_h = [None] * NBUF
    gather_h = [None] * NBUF
    pos_h[0] = start_pos(0)
    idx_h = []
    for b in range(B):
        idx_h.append(pltpu.async_copy(
            x_hbm.at[pl.ds(b * SEQ + wpos, POS_PER_W)],
            idx_v.at[pl.ds(b * POS_PER_W, POS_PER_W)],
            isem,
        ))
    for h in idx_h:
        h.wait()
    for t in range(DEPTH):
        gather_h[t % NBUF] = start_gather(t)

    for t, (ph, b) in enumerate(_TASKS):
        cur = t % NBUF
        if b == 0 and ph + 1 < NPG:
            pos_h[(ph + 1) % 2] = start_pos(ph + 1)
        if t + DEPTH < len(_TASKS):
            nb = (t + DEPTH) % NBUF
            if out_h[nb] is not None:
                out_h[nb].wait()
                out_h[nb] = None
            gather_h[nb] = start_gather(t + DEPTH)
        gather_h[cur].wait()
        if b == 0:
            pos_h[ph % 2].wait()

        wbuf, pbuf = word[cur], pos[ph % 2]

        def add_body(k, carry):
            r = k // GROUPS
            g = k % GROUPS
            sl = pl.ds(g * LANES, LANES)
            plsc.addupdate(wbuf.at[r, sl], pbuf[r, sl])
            return carry

        lax.fori_loop(0, PC * GROUPS, add_body, 0, unroll=4)
        out_h[cur] = pltpu.async_copy(
            word[cur], out_hbm.at[pl.ds(b * SEQ + wpos + ph * PC, PC)],
            osem[cur],
        )

    for h in out_h:
        if h is not None:
            h.wait()


def kernel(x, word_table, pos_table):
    x_flat = x.reshape(N).astype(jnp.int32)
    out = _embed(x_flat, word_table, pos_table)
    return out.reshape(1, B, SEQ, DMODEL)
